# baseline (speedup 1.0000x reference)
import functools

import jax
import jax.numpy as jnp
from jax import lax
from jax.experimental import pallas as pl
from jax.experimental.pallas import tpu as pltpu
from jax.experimental.pallas import tpu_sc as plsc

NUM_CORES = 2
NUM_SUBCORES = 16
NW = NUM_CORES * NUM_SUBCORES
LANES = 16

SEQ_SPLIT = 3072
TC_BLOCK = 512
SC_CHUNK = 4
SC_DEPTH = 2
SC_UNROLL = 8


def _tc_add(x_ref, emb_ref, out_ref):
    out_ref[...] = x_ref[...] + emb_ref[...][None, :, :]


def _tc_part(x, emb_weight):
    batch, seq_len, dim = x.shape
    off_blocks = (emb_weight.shape[0] - seq_len) // TC_BLOCK
    return pl.pallas_call(
        _tc_add,
        grid=(SEQ_SPLIT // TC_BLOCK,),
        in_specs=[
            pl.BlockSpec((batch, TC_BLOCK, dim), lambda s: (0, s, 0)),
            pl.BlockSpec((TC_BLOCK, dim), lambda s: (s + off_blocks, 0)),
        ],
        out_specs=pl.BlockSpec((batch, TC_BLOCK, dim), lambda s: (0, s, 0)),
        out_shape=jax.ShapeDtypeStruct((batch, SEQ_SPLIT, dim), x.dtype),
    )(x, emb_weight)


def _sc_body(x_hbm, emb_hbm, out_hbm, xbuf, ebuf, obuf, xsem, esem, osem):
    batch, seq_len, dim = x_hbm.shape
    off = emb_hbm.shape[0] - seq_len
    sc_rows = seq_len - SEQ_SPLIT
    per_w = sc_rows // NW
    n = per_w // SC_CHUNK
    wid = lax.axis_index("s") * NUM_CORES + lax.axis_index("c")
    base = SEQ_SPLIT + wid * per_w

    def in_copies(i, slot):
        s0 = base + i * SC_CHUNK
        return [
            pltpu.make_async_copy(
                emb_hbm.at[pl.ds(off + s0, SC_CHUNK), :], ebuf.at[slot],
                esem.at[slot]),
            pltpu.make_async_copy(
                x_hbm.at[:, pl.ds(s0, SC_CHUNK), :], xbuf.at[slot],
                xsem.at[slot]),
        ]

    def out_copies(i, slot):
        s0 = base - SEQ_SPLIT + i * SC_CHUNK
        return [pltpu.make_async_copy(
            obuf.at[slot], out_hbm.at[:, pl.ds(s0, SC_CHUNK), :],
            osem.at[slot])]

    for s in range(SC_DEPTH):
        for c in in_copies(s, s):
            c.start()

    def group(g, carry):
        for slot in range(SC_DEPTH):
            i = g * SC_DEPTH + slot
            for c in in_copies(i, slot):
                c.wait()

            @pl.when(i >= SC_DEPTH)
            def _wait_out(i=i, slot=slot):
                for c in out_copies(i - SC_DEPTH, slot):
                    c.wait()

            @plsc.parallel_loop(0, dim, LANES, unroll=SC_UNROLL)
            def _col(d0, slot=slot):
                for s in range(SC_CHUNK):
                    e = ebuf[slot, s, pl.ds(d0, LANES)]
                    for b in range(batch):
                        obuf[slot, b, s, pl.ds(d0, LANES)] = (
                            xbuf[slot, b, s, pl.ds(d0, LANES)] + e)

            for c in out_copies(i, slot):
                c.start()

            @pl.when(i + SC_DEPTH < n)
            def _next_in(i=i, slot=slot):
                for c in in_copies(i + SC_DEPTH, slot):
                    c.start()

        return carry

    lax.fori_loop(0, n // SC_DEPTH, group, 0)

    for k in range(max(0, n - SC_DEPTH), n):
        for c in out_copies(k, k % SC_DEPTH):
            c.wait()


def _sc_part(x, emb_weight):
    batch, seq_len, dim = x.shape
    run = functools.partial(
        pl.kernel,
        out_type=jax.ShapeDtypeStruct((batch, seq_len - SEQ_SPLIT, dim),
                                      x.dtype),
        mesh=plsc.VectorSubcoreMesh(
            core_axis_name="c", subcore_axis_name="s",
            num_cores=NUM_CORES, num_subcores=NUM_SUBCORES),
        scratch_types=[
            pltpu.VMEM((SC_DEPTH, batch, SC_CHUNK, dim), jnp.float32),
            pltpu.VMEM((SC_DEPTH, SC_CHUNK, dim), jnp.float32),
            pltpu.VMEM((SC_DEPTH, batch, SC_CHUNK, dim), jnp.float32),
            pltpu.SemaphoreType.DMA((SC_DEPTH,)),
            pltpu.SemaphoreType.DMA((SC_DEPTH,)),
            pltpu.SemaphoreType.DMA((SC_DEPTH,)),
        ],
    )(_sc_body)
    return run(x, emb_weight)


def kernel(x, emb_weight):
    tc_out = _tc_part(x, emb_weight)
    sc_out = _sc_part(x, emb_weight)
    return jnp.concatenate([tc_out, sc_out], axis=1)


# speedup vs baseline: 2.2158x; 2.2158x over previous
import jax
import jax.numpy as jnp
from jax.experimental import pallas as pl

SEQ_BLOCK = 512


def _add_kernel(x_ref, emb_ref, out_ref):
    out_ref[...] = x_ref[...] + emb_ref[...][None, :, :]


def kernel(x, emb_weight):
    batch, seq_len, dim = x.shape
    offset_blocks = (emb_weight.shape[0] - seq_len) // SEQ_BLOCK
    num_blocks = seq_len // SEQ_BLOCK
    return pl.pallas_call(
        _add_kernel,
        grid=(num_blocks,),
        in_specs=[
            pl.BlockSpec((batch, SEQ_BLOCK, dim), lambda s: (0, s, 0)),
            pl.BlockSpec((SEQ_BLOCK, dim), lambda s: (s + offset_blocks, 0)),
        ],
        out_specs=pl.BlockSpec((batch, SEQ_BLOCK, dim), lambda s: (0, s, 0)),
        out_shape=jax.ShapeDtypeStruct(x.shape, x.dtype),
    )(x, emb_weight)


# speedup vs baseline: 2.2228x; 1.0031x over previous
import jax
from jax.experimental import pallas as pl

SEQ_BLOCK = 512


def _add_kernel(x_ref, emb_ref, out_ref):
    out_ref[...] = x_ref[...] + emb_ref[...][None, :, :]


def kernel(x, emb_weight):
    batch, seq_len, dim = x.shape
    offset_blocks = (emb_weight.shape[0] - seq_len) // SEQ_BLOCK
    num_blocks = seq_len // SEQ_BLOCK
    return pl.pallas_call(
        _add_kernel,
        grid=(num_blocks,),
        in_specs=[
            pl.BlockSpec((batch, SEQ_BLOCK, dim), lambda s: (0, s, 0)),
            pl.BlockSpec((SEQ_BLOCK, dim), lambda s: (s + offset_blocks, 0)),
        ],
        out_specs=pl.BlockSpec((batch, SEQ_BLOCK, dim), lambda s: (0, s, 0)),
        out_shape=jax.ShapeDtypeStruct(x.shape, x.dtype),
    )(x, emb_weight)
